# baseline (speedup 1.0000x reference)
import functools

import jax
import jax.numpy as jnp
import numpy as np
from jax import lax
from jax.experimental import pallas as pl
from jax.experimental.pallas import tpu as pltpu
from jax.experimental.pallas import tpu_sc as plsc

_N_DIM = 3
_N_LEVELS = 16
_LOG2_HASHMAP = 19
_N_FEATURE = 2
_RES_COARSE = 16
_RES_FINE = 2048
_LEVEL_SCALE = float(np.exp2(np.log2(_RES_FINE / _RES_COARSE) / (_N_LEVELS - 1)))

_K2 = np.int32(np.uint32(2654435761).astype(np.int64) - (1 << 32))
_K3 = np.int32(805459861)


def _level_metas():
    metas = []
    offset = 0
    max_params = 2 ** _LOG2_HASHMAP
    for i in range(_N_LEVELS):
        res = int(np.ceil(_RES_COARSE * _LEVEL_SCALE ** i))
        params = min(max_params, res ** _N_DIM)
        dense = res ** _N_DIM <= params
        metas.append((res, params, offset, dense))
        offset += params
    return metas


_METAS = _level_metas()
_N_STAGED = 3
_STAGED_ROWS = sum(m[1] for m in _METAS[:_N_STAGED])

_NC = 2
_NS = 16
_NW = _NC * _NS
_L = 16
_NOUT = _N_LEVELS * _N_FEATURE


@functools.partial(jax.jit, static_argnames=("n_points", "chunk"))
def _hash_encode_sc(inputs_t, emb_packed, n_points, chunk):
    pts_per_w = n_points // _NW
    n_chunks = pts_per_w // chunk
    groups = chunk // _L
    rows_per_chunk = 8 * chunk

    mesh = plsc.VectorSubcoreMesh(
        core_axis_name="c", subcore_axis_name="s", num_cores=_NC, num_subcores=_NS
    )

    @functools.partial(
        pl.kernel,
        out_type=jax.ShapeDtypeStruct((n_points * _NOUT,), jnp.float32),
        mesh=mesh,
        compiler_params=pltpu.CompilerParams(needs_layout_passes=False),
        scratch_types=[
            pltpu.VMEM((_N_DIM * chunk,), jnp.float32),
            pltpu.VMEM((_N_DIM * chunk,), jnp.float32),
            pltpu.VMEM((2, groups, 8 * _L), jnp.int32),
            pltpu.VMEM((2, groups * 8 * _L), jnp.float32),
            pltpu.VMEM((2, rows_per_chunk), jnp.int32),
            pltpu.VMEM((2 * chunk * _NOUT,), jnp.float32),
            pltpu.VMEM((_STAGED_ROWS,), jnp.int32),
            pltpu.SemaphoreType.DMA,
            pltpu.SemaphoreType.DMA,
            pltpu.SemaphoreType.DMA,
        ],
    )
    def run(in_hbm, emb_hbm, out_hbm, in_v, raw_v, idx_v, w_v, rows_v, out_v,
            tab_v, sem0, sem1, sem_out):
        pltpu.sync_copy(emb_hbm.at[pl.ds(0, _STAGED_ROWS)], tab_v)
        wid = lax.axis_index("s") * _NC + lax.axis_index("c")
        iota16 = lax.iota(jnp.int32, _L)
        sems = (sem0, sem1)

        def chunk_body(g, carry):
            base = wid * pts_per_w + g * chunk
            gb = g % 2
            @pl.when(g >= 2)
            def _():
                pltpu.make_async_copy(
                    out_hbm.at[pl.ds(0, chunk * _NOUT)],
                    out_v.at[pl.ds(0, chunk * _NOUT)],
                    sem_out,
                ).wait()

            pltpu.sync_copy(
                in_hbm.at[pl.ds(base * _N_DIM, chunk * _N_DIM)], raw_v
            )

            def transpose_grp(i, carry):
                pid3 = (i * _L + iota16) * _N_DIM
                for d in range(_N_DIM):
                    in_v[pl.ds(d * chunk + i * _L, _L)] = plsc.load_gather(
                        raw_v, [pid3 + d])
                return carry

            lax.fori_loop(0, groups, transpose_grp, 0, unroll=False)

            def compute_level(li, b):
                res, params, offset, dense = _METAS[li]
                resf = jnp.float32(res - 1)
                rmax = jnp.int32(res - 1)

                def compute_grp(i, carry):
                    x = in_v[pl.ds(i * _L, _L)]
                    y = in_v[pl.ds(chunk + i * _L, _L)]
                    z = in_v[pl.ds(2 * chunk + i * _L, _L)]

                    px = x * resf
                    py = y * resf
                    pz = z * resf
                    ix0 = jnp.minimum(px.astype(jnp.int32), rmax)
                    iy0 = jnp.minimum(py.astype(jnp.int32), rmax)
                    iz0 = jnp.minimum(pz.astype(jnp.int32), rmax)
                    fx = px - ix0.astype(jnp.float32)
                    fy = py - iy0.astype(jnp.float32)
                    fz = pz - iz0.astype(jnp.float32)
                    ix1 = jnp.minimum(ix0 + 1, rmax)
                    iy1 = jnp.minimum(iy0 + 1, rmax)
                    iz1 = jnp.minimum(iz0 + 1, rmax)

                    if dense:
                        ax = (ix0, ix1)
                        ay = (iy0 * res, iy1 * res)
                        az = (iz0 * (res * res), iz1 * (res * res))

                        def cidx(bx, by, bz):
                            return ax[bx] + ay[by] + az[bz] + jnp.int32(offset)
                    else:
                        mask = jnp.int32(params - 1)
                        ax = (ix0, ix1)
                        ay = (iy0 * _K2, iy1 * _K2)
                        az = (iz0 * _K3, iz1 * _K3)

                        def cidx(bx, by, bz):
                            h = ax[bx] ^ ay[by] ^ az[bz]
                            return (h & mask) + jnp.int32(offset)

                    gx = (1.0 - fx, fx)
                    gy = (1.0 - fy, fy)
                    gz = (1.0 - fz, fz)
                    wxy = {(bx, by): gx[bx] * gy[by]
                           for bx in (0, 1) for by in (0, 1)}

                    for c in range(8):
                        bx, by, bz = c & 1, (c >> 1) & 1, (c >> 2) & 1
                        idx_v[b, i, pl.ds(c * _L, _L)] = cidx(bx, by, bz)
                        w_v[b, pl.ds(i * (8 * _L) + c * _L, _L)] = (
                            wxy[(bx, by)] * gz[bz])
                    pltpu.async_copy(
                        emb_hbm.at[idx_v.at[b, i]],
                        rows_v.at[b, pl.ds(i * (8 * _L), 8 * _L)],
                        sems[b],
                    )
                    return carry

                lax.fori_loop(0, groups, compute_grp, 0, unroll=False)

            def drain_acc_level(li, b):
                pltpu.make_async_copy(
                    emb_hbm.at[pl.ds(0, rows_per_chunk)], rows_v.at[b], sems[b]
                ).wait()

                def acc_grp(i, carry):
                    rb = i * (8 * _L)
                    acc0 = jnp.zeros((_L,), jnp.float32)
                    acc1 = jnp.zeros((_L,), jnp.float32)
                    for c in range(8):
                        vi = rows_v[b, pl.ds(rb + c * _L, _L)]
                        vb = plsc.bitcast(vi, jnp.bfloat16)
                        f0, f1 = plsc.unpack(
                            vb, format=plsc.PackFormat.INTERLEAVED)
                        w = w_v[b, pl.ds(rb + c * _L, _L)]
                        acc0 = acc0 + w * f0
                        acc1 = acc1 + w * f1
                    oid = (gb * (chunk * _NOUT)
                           + (i * _L + iota16) * _NOUT + (2 * li))
                    plsc.store_scatter(out_v, [oid], acc0)
                    plsc.store_scatter(out_v, [oid + 1], acc1)
                    return carry

                lax.fori_loop(0, groups, acc_grp, 0, unroll=False)

            def fused_level(li):
                res, params, offset, dense = _METAS[li]
                assert dense
                resf = jnp.float32(res - 1)
                rmax = jnp.int32(res - 1)

                def fused_grp(i, carry):
                    x = in_v[pl.ds(i * _L, _L)]
                    y = in_v[pl.ds(chunk + i * _L, _L)]
                    z = in_v[pl.ds(2 * chunk + i * _L, _L)]

                    px = x * resf
                    py = y * resf
                    pz = z * resf
                    ix0 = jnp.minimum(px.astype(jnp.int32), rmax)
                    iy0 = jnp.minimum(py.astype(jnp.int32), rmax)
                    iz0 = jnp.minimum(pz.astype(jnp.int32), rmax)
                    fx = px - ix0.astype(jnp.float32)
                    fy = py - iy0.astype(jnp.float32)
                    fz = pz - iz0.astype(jnp.float32)
                    ax = (ix0, jnp.minimum(ix0 + 1, rmax))
                    ay = (iy0 * res, jnp.minimum(iy0 + 1, rmax) * res)
                    az = (iz0 * (res * res),
                          jnp.minimum(iz0 + 1, rmax) * (res * res))
                    gx = (1.0 - fx, fx)
                    gy = (1.0 - fy, fy)
                    gz = (1.0 - fz, fz)
                    wxy = {(bx, by): gx[bx] * gy[by]
                           for bx in (0, 1) for by in (0, 1)}

                    acc0 = jnp.zeros((_L,), jnp.float32)
                    acc1 = jnp.zeros((_L,), jnp.float32)
                    for c in range(8):
                        bx, by, bz = c & 1, (c >> 1) & 1, (c >> 2) & 1
                        cid = ax[bx] + ay[by] + az[bz] + jnp.int32(offset)
                        vi = plsc.load_gather(tab_v, [cid])
                        vb = plsc.bitcast(vi, jnp.bfloat16)
                        f0, f1 = plsc.unpack(
                            vb, format=plsc.PackFormat.INTERLEAVED)
                        w = wxy[(bx, by)] * gz[bz]
                        acc0 = acc0 + w * f0
                        acc1 = acc1 + w * f1
                    oid = (gb * (chunk * _NOUT)
                           + (i * _L + iota16) * _NOUT + (2 * li))
                    plsc.store_scatter(out_v, [oid], acc0)
                    plsc.store_scatter(out_v, [oid + 1], acc1)
                    return carry

                lax.fori_loop(0, groups, fused_grp, 0, unroll=False)

            compute_level(_N_STAGED, _N_STAGED % 2)
            for li in range(_N_STAGED):
                fused_level(li)
            for li in range(_N_STAGED, _N_LEVELS):
                if li + 1 < _N_LEVELS:
                    compute_level(li + 1, (li + 1) % 2)
                drain_acc_level(li, li % 2)

            pltpu.async_copy(
                out_v.at[pl.ds(gb * (chunk * _NOUT), chunk * _NOUT)],
                out_hbm.at[pl.ds(base * _NOUT, chunk * _NOUT)],
                sem_out,
            )
            return carry

        lax.fori_loop(0, n_chunks, chunk_body, 0, unroll=False)
        for _ in range(min(2, n_chunks)):
            pltpu.make_async_copy(
                out_hbm.at[pl.ds(0, chunk * _NOUT)],
                out_v.at[pl.ds(0, chunk * _NOUT)],
                sem_out,
            ).wait()

    out = run(inputs_t, emb_packed)
    return out.reshape(n_points, _NOUT)


def kernel(inputs, embeddings):
    inputs = jnp.reshape(inputs, (-1, _N_DIM))
    n = inputs.shape[0]
    emb_packed = lax.bitcast_convert_type(
        embeddings.astype(jnp.bfloat16), jnp.int32
    )
    return _hash_encode_sc(inputs.reshape(-1), emb_packed, n, 512)


# speedup vs baseline: 1.1684x; 1.1684x over previous
import functools

import jax
import jax.numpy as jnp
import numpy as np
from jax import lax
from jax.experimental import pallas as pl
from jax.experimental.pallas import tpu as pltpu
from jax.experimental.pallas import tpu_sc as plsc

_N_DIM = 3
_N_LEVELS = 16
_LOG2_HASHMAP = 19
_N_FEATURE = 2
_RES_COARSE = 16
_RES_FINE = 2048
_LEVEL_SCALE = float(np.exp2(np.log2(_RES_FINE / _RES_COARSE) / (_N_LEVELS - 1)))

_K2 = np.int32(np.uint32(2654435761).astype(np.int64) - (1 << 32))
_K3 = np.int32(805459861)


def _level_metas():
    metas = []
    offset = 0
    max_params = 2 ** _LOG2_HASHMAP
    for i in range(_N_LEVELS):
        res = int(np.ceil(_RES_COARSE * _LEVEL_SCALE ** i))
        params = min(max_params, res ** _N_DIM)
        dense = res ** _N_DIM <= params
        metas.append((res, params, offset, dense))
        offset += params
    return metas


_METAS = _level_metas()
_N_STAGED = 3
_STAGED_ROWS = sum(m[1] for m in _METAS[:_N_STAGED])

_NC = 2
_NS = 16
_NW = _NC * _NS
_L = 16
_NOUT = _N_LEVELS * _N_FEATURE


@functools.partial(jax.jit, static_argnames=("n_points", "chunk"))
def _hash_encode_sc(inputs_t, emb_packed, n_points, chunk):
    pts_per_w = n_points // _NW
    n_chunks = pts_per_w // chunk
    groups = chunk // _L
    rows_per_chunk = 8 * chunk

    mesh = plsc.VectorSubcoreMesh(
        core_axis_name="c", subcore_axis_name="s", num_cores=_NC, num_subcores=_NS
    )

    @functools.partial(
        pl.kernel,
        out_type=jax.ShapeDtypeStruct((n_points * _NOUT,), jnp.float32),
        mesh=mesh,
        compiler_params=pltpu.CompilerParams(needs_layout_passes=False),
        scratch_types=[
            pltpu.VMEM((_N_DIM * chunk,), jnp.float32),
            pltpu.VMEM((2, groups, 8 * _L), jnp.int32),
            pltpu.VMEM((2, groups * 8 * _L), jnp.float32),
            pltpu.VMEM((2, rows_per_chunk), jnp.int32),
            pltpu.VMEM((2 * chunk * _NOUT,), jnp.float32),
            pltpu.VMEM((_STAGED_ROWS,), jnp.int32),
            pltpu.SemaphoreType.DMA,
            pltpu.SemaphoreType.DMA,
            pltpu.SemaphoreType.DMA,
        ],
    )
    def run(in_hbm, emb_hbm, out_hbm, in_v, idx_v, w_v, rows_v, out_v,
            tab_v, sem0, sem1, sem_out):
        pltpu.sync_copy(emb_hbm.at[pl.ds(0, _STAGED_ROWS)], tab_v)
        wid = lax.axis_index("s") * _NC + lax.axis_index("c")
        iota16 = lax.iota(jnp.int32, _L)
        sems = (sem0, sem1)

        def chunk_body(g, carry):
            base = wid * pts_per_w + g * chunk
            gb = g % 2
            @pl.when(g >= 2)
            def _():
                pltpu.make_async_copy(
                    out_hbm.at[pl.ds(0, chunk * _NOUT)],
                    out_v.at[pl.ds(0, chunk * _NOUT)],
                    sem_out,
                ).wait()

            for d in range(_N_DIM):
                pltpu.sync_copy(
                    in_hbm.at[pl.ds(d * n_points + base, chunk)],
                    in_v.at[pl.ds(d * chunk, chunk)],
                )

            def compute_level(li, b):
                res, params, offset, dense = _METAS[li]
                resf = jnp.float32(res - 1)
                rmax = jnp.int32(res - 1)

                def compute_grp(i, carry):
                    x = in_v[pl.ds(i * _L, _L)]
                    y = in_v[pl.ds(chunk + i * _L, _L)]
                    z = in_v[pl.ds(2 * chunk + i * _L, _L)]

                    px = x * resf
                    py = y * resf
                    pz = z * resf
                    ix0 = jnp.minimum(px.astype(jnp.int32), rmax)
                    iy0 = jnp.minimum(py.astype(jnp.int32), rmax)
                    iz0 = jnp.minimum(pz.astype(jnp.int32), rmax)
                    fx = px - ix0.astype(jnp.float32)
                    fy = py - iy0.astype(jnp.float32)
                    fz = pz - iz0.astype(jnp.float32)
                    ix1 = jnp.minimum(ix0 + 1, rmax)
                    iy1 = jnp.minimum(iy0 + 1, rmax)
                    iz1 = jnp.minimum(iz0 + 1, rmax)

                    if dense:
                        ax = (ix0, ix1)
                        ay = (iy0 * res, iy1 * res)
                        az = (iz0 * (res * res), iz1 * (res * res))

                        def cidx(bx, by, bz):
                            return ax[bx] + ay[by] + az[bz] + jnp.int32(offset)
                    else:
                        mask = jnp.int32(params - 1)
                        ax = (ix0, ix1)
                        ay = (iy0 * _K2, iy1 * _K2)
                        az = (iz0 * _K3, iz1 * _K3)

                        def cidx(bx, by, bz):
                            h = ax[bx] ^ ay[by] ^ az[bz]
                            return (h & mask) + jnp.int32(offset)

                    gx = (1.0 - fx, fx)
                    gy = (1.0 - fy, fy)
                    gz = (1.0 - fz, fz)
                    wxy = {(bx, by): gx[bx] * gy[by]
                           for bx in (0, 1) for by in (0, 1)}

                    for c in range(8):
                        bx, by, bz = c & 1, (c >> 1) & 1, (c >> 2) & 1
                        idx_v[b, i, pl.ds(c * _L, _L)] = cidx(bx, by, bz)
                        w_v[b, pl.ds(i * (8 * _L) + c * _L, _L)] = (
                            wxy[(bx, by)] * gz[bz])
                    pltpu.async_copy(
                        emb_hbm.at[idx_v.at[b, i]],
                        rows_v.at[b, pl.ds(i * (8 * _L), 8 * _L)],
                        sems[b],
                    )
                    return carry

                lax.fori_loop(0, groups, compute_grp, 0, unroll=False)

            def drain_acc_level(li, b):
                pltpu.make_async_copy(
                    emb_hbm.at[pl.ds(0, rows_per_chunk)], rows_v.at[b], sems[b]
                ).wait()

                def acc_grp(i, carry):
                    rb = i * (8 * _L)
                    acc0 = jnp.zeros((_L,), jnp.float32)
                    acc1 = jnp.zeros((_L,), jnp.float32)
                    for c in range(8):
                        vi = rows_v[b, pl.ds(rb + c * _L, _L)]
                        vb = plsc.bitcast(vi, jnp.bfloat16)
                        f0, f1 = plsc.unpack(
                            vb, format=plsc.PackFormat.INTERLEAVED)
                        w = w_v[b, pl.ds(rb + c * _L, _L)]
                        acc0 = acc0 + w * f0
                        acc1 = acc1 + w * f1
                    oid = (gb * (chunk * _NOUT)
                           + (i * _L + iota16) * _NOUT + (2 * li))
                    plsc.store_scatter(out_v, [oid], acc0)
                    plsc.store_scatter(out_v, [oid + 1], acc1)
                    return carry

                lax.fori_loop(0, groups, acc_grp, 0, unroll=False)

            def fused_level(li):
                res, params, offset, dense = _METAS[li]
                assert dense
                resf = jnp.float32(res - 1)
                rmax = jnp.int32(res - 1)

                def fused_grp(i, carry):
                    x = in_v[pl.ds(i * _L, _L)]
                    y = in_v[pl.ds(chunk + i * _L, _L)]
                    z = in_v[pl.ds(2 * chunk + i * _L, _L)]

                    px = x * resf
                    py = y * resf
                    pz = z * resf
                    ix0 = jnp.minimum(px.astype(jnp.int32), rmax)
                    iy0 = jnp.minimum(py.astype(jnp.int32), rmax)
                    iz0 = jnp.minimum(pz.astype(jnp.int32), rmax)
                    fx = px - ix0.astype(jnp.float32)
                    fy = py - iy0.astype(jnp.float32)
                    fz = pz - iz0.astype(jnp.float32)
                    ax = (ix0, jnp.minimum(ix0 + 1, rmax))
                    ay = (iy0 * res, jnp.minimum(iy0 + 1, rmax) * res)
                    az = (iz0 * (res * res),
                          jnp.minimum(iz0 + 1, rmax) * (res * res))
                    gx = (1.0 - fx, fx)
                    gy = (1.0 - fy, fy)
                    gz = (1.0 - fz, fz)
                    wxy = {(bx, by): gx[bx] * gy[by]
                           for bx in (0, 1) for by in (0, 1)}

                    acc0 = jnp.zeros((_L,), jnp.float32)
                    acc1 = jnp.zeros((_L,), jnp.float32)
                    for c in range(8):
                        bx, by, bz = c & 1, (c >> 1) & 1, (c >> 2) & 1
                        cid = ax[bx] + ay[by] + az[bz] + jnp.int32(offset)
                        vi = plsc.load_gather(tab_v, [cid])
                        vb = plsc.bitcast(vi, jnp.bfloat16)
                        f0, f1 = plsc.unpack(
                            vb, format=plsc.PackFormat.INTERLEAVED)
                        w = wxy[(bx, by)] * gz[bz]
                        acc0 = acc0 + w * f0
                        acc1 = acc1 + w * f1
                    oid = (gb * (chunk * _NOUT)
                           + (i * _L + iota16) * _NOUT + (2 * li))
                    plsc.store_scatter(out_v, [oid], acc0)
                    plsc.store_scatter(out_v, [oid + 1], acc1)
                    return carry

                lax.fori_loop(0, groups, fused_grp, 0, unroll=False)

            compute_level(_N_STAGED, _N_STAGED % 2)
            for li in range(_N_STAGED):
                fused_level(li)
            for li in range(_N_STAGED, _N_LEVELS):
                if li + 1 < _N_LEVELS:
                    compute_level(li + 1, (li + 1) % 2)
                drain_acc_level(li, li % 2)

            pltpu.async_copy(
                out_v.at[pl.ds(gb * (chunk * _NOUT), chunk * _NOUT)],
                out_hbm.at[pl.ds(base * _NOUT, chunk * _NOUT)],
                sem_out,
            )
            return carry

        lax.fori_loop(0, n_chunks, chunk_body, 0, unroll=False)
        for _ in range(min(2, n_chunks)):
            pltpu.make_async_copy(
                out_hbm.at[pl.ds(0, chunk * _NOUT)],
                out_v.at[pl.ds(0, chunk * _NOUT)],
                sem_out,
            ).wait()

    out = run(inputs_t, emb_packed)
    return out.reshape(n_points, _NOUT)


def kernel(inputs, embeddings):
    inputs = jnp.reshape(inputs, (-1, _N_DIM))
    n = inputs.shape[0]
    emb_packed = lax.bitcast_convert_type(
        embeddings.astype(jnp.bfloat16), jnp.int32
    )
    return _hash_encode_sc(inputs.T.reshape(-1), emb_packed, n, 512)
